# Initial kernel scaffold; baseline (speedup 1.0000x reference)
import jax
import jax.numpy as jnp
from jax.experimental import pallas as pl


def kernel(image_output, slic_output):
    raise NotImplementedError("write your pallas kernel here")


# speedup vs baseline: 9.9426x; 9.9426x over previous
import functools

import jax
import jax.numpy as jnp
from jax import lax
from jax.experimental import pallas as pl
from jax.experimental.pallas import tpu as pltpu
from jax.experimental.pallas import tpu_sc as plsc

B = 4
HW = 224 * 224
C = 96
NSEG = 100
NPIX = B * HW
NC = 2
NS = 16
NW = NC * NS
PW = NPIX // NW
CH = 128
NCHUNK = PW // CH
PARTS_PER_B = NW // B
ACC_ROWS = 512


def _sc_body(img_hbm, slic_hbm, psum_hbm, pcnt_hbm,
             slic_v, idx_v, img_v, ind_v, acc_s, acc_c, sem_in):
    c = lax.axis_index("c")
    s = lax.axis_index("s")
    wid = s * NC + c
    b = wid // PARTS_PER_B
    part = wid % PARTS_PER_B
    base = b * HW + part * PW

    pltpu.sync_copy(slic_hbm.at[pl.ds(base, PW)], slic_v)

    def idx_body(j, _):
        for k in range(CH // 16):
            v = slic_v[pl.ds(j * CH + k * 16, 16)]
            ix = jnp.where(v == 0, 400 + b, v * 4 + (b - 4))
            idx_v[j, pl.ds(k * 16, 16)] = ix
        return 0
    lax.fori_loop(0, NCHUNK, idx_body, 0)

    zero = jnp.zeros((16,), jnp.float32)

    def z_body(r, _):
        for k in range(C // 16):
            img_v[0, r, pl.ds(k * 16, 16)] = zero
        return 0
    lax.fori_loop(0, CH, z_body, 0)

    @pl.when(s < 8)
    def _():
        row0 = (s % 4) * CH

        @pl.when(s < 4)
        def _():
            pltpu.sync_copy(img_v.at[0], acc_s.at[pl.ds(row0, CH)])

        @pl.when(s >= 4)
        def _():
            pltpu.sync_copy(img_v.at[0], acc_c.at[pl.ds(row0, CH)])

    plsc.subcore_barrier()

    pltpu.async_copy(img_hbm.at[pl.ds(base, CH)], img_v.at[0], sem_in.at[0])

    def chunk_body(j, _):
        buf = j % 2
        nbuf = (j + 1) % 2

        @pl.when(j < NCHUNK - 1)
        def _():
            pltpu.async_copy(img_hbm.at[pl.ds(base + (j + 1) * CH, CH)],
                             img_v.at[nbuf], sem_in.at[nbuf])

        pltpu.make_async_copy(img_hbm.at[pl.ds(base, CH)], img_v.at[buf],
                              sem_in.at[buf]).wait()

        def ind_body(r, _):
            for k in range(C // 16):
                x = img_v[buf, r, pl.ds(k * 16, 16)]
                one = jnp.full((16,), 1.0, jnp.float32)
                zer = jnp.zeros((16,), jnp.float32)
                ind_v[r, pl.ds(k * 16, 16)] = jnp.where(x != 0.0, one, zer)
            return 0
        lax.fori_loop(0, CH, ind_body, 0)

        pltpu.sync_copy(img_v.at[buf], acc_s.at[idx_v.at[j]], add=True)
        pltpu.sync_copy(ind_v, acc_c.at[idx_v.at[j]], add=True)
        return 0

    lax.fori_loop(0, NCHUNK, chunk_body, 0)

    plsc.subcore_barrier()

    @pl.when(s == 0)
    def _():
        pltpu.sync_copy(acc_s, psum_hbm.at[c])
        pltpu.sync_copy(acc_c, pcnt_hbm.at[c])


@jax.jit
def _sc_call(img2d, slic1d):
    mesh = plsc.VectorSubcoreMesh(core_axis_name="c", subcore_axis_name="s")
    f = pl.kernel(
        _sc_body,
        out_type=(
            jax.ShapeDtypeStruct((NC, ACC_ROWS, C), jnp.float32),
            jax.ShapeDtypeStruct((NC, ACC_ROWS, C), jnp.float32),
        ),
        mesh=mesh,
        scratch_types=[
            pltpu.VMEM((PW,), jnp.int32),
            pltpu.VMEM((NCHUNK, CH), jnp.int32),
            pltpu.VMEM((2, CH, C), jnp.float32),
            pltpu.VMEM((CH, C), jnp.float32),
            pltpu.VMEM_SHARED((ACC_ROWS, C), jnp.float32),
            pltpu.VMEM_SHARED((ACC_ROWS, C), jnp.float32),
            pltpu.SemaphoreType.DMA((2,)),
        ],
    )
    return f(img2d, slic1d)


def _combine_body(ps_ref, pc_ref, o_ref):
    ssum = ps_ref[0, 0:NSEG * B, :] + ps_ref[1, 0:NSEG * B, :]
    scnt = pc_ref[0, 0:NSEG * B, :] + pc_ref[1, 0:NSEG * B, :]
    o_ref[...] = ssum / scnt


@jax.jit
def _combine_call(psum, pcnt):
    return pl.pallas_call(
        _combine_body,
        out_shape=jax.ShapeDtypeStruct((NSEG * B, C), jnp.float32),
    )(psum, pcnt)


def kernel(image_output, slic_output):
    img2d = image_output.reshape(NPIX, C)
    slic1d = slic_output.reshape(NPIX)
    psum, pcnt = _sc_call(img2d, slic1d)
    out2d = _combine_call(psum, pcnt)
    return out2d.reshape(NSEG, B, C)
